# baseline (speedup 1.0000x reference)
import functools

import jax
import jax.numpy as jnp
from jax import lax
from jax.experimental import pallas as pl
from jax.experimental.pallas import tpu as pltpu
from jax.experimental.pallas import tpu_sc as plsc

VOCAB = 1000000
EMBED = 32
BATCH = 16384

_VB = 7813
_VPAD = _VB * 128
_STRIDE = 129

_info = plsc.get_sparse_core_info()
_NC, _NS, _L = _info.num_cores, _info.num_subcores, _info.num_lanes
_NW = _NC * _NS
_BPW = BATCH // _NW
_SLOTS = 8


def _detile_kernel(tblT_hbm, out_hbm, in_v, tr_v, sem_i, sem_o):
    wid = lax.axis_index("s") * _NC + lax.axis_index("c")
    n_i = jnp.int32((_VB + _NW - 1) // _NW)
    n_i = jnp.where(wid < jnp.int32(_VB % _NW), n_i, n_i - 1)
    n_chunks = (n_i + (_SLOTS - 1)) // _SLOTS
    lanes = lax.iota(jnp.int32, 16)

    def in_copy(b, vb, g):
        return pltpu.make_async_copy(
            tblT_hbm.at[pl.ds(g * 8, 8), pl.ds(vb * 128, 128)],
            in_v.at[pl.ds(b * 32 + g * 8, 8), pl.ds(0, 128)], sem_i)

    def out_copy(b, vb):
        return pltpu.make_async_copy(
            tr_v.at[pl.ds(b * 4096, 4096)],
            out_hbm.at[pl.ds(vb * 4096, 4096)], sem_o)

    def transpose_slot(b):
        def tk(kk, c):
            for u in range(8):
                row = jnp.int32(b * 32 + (u & 1) * 16) + lanes
                col = jnp.zeros((16,), jnp.int32) + (kk * 4 + (u >> 1))
                x = plsc.load_gather(in_v, [row, col])
                tr_v[pl.ds(b * 4096 + kk * 128 + u * 16, 16)] = x
            return c

        lax.fori_loop(0, 32, tk, 0)

    def chunk_body(ci, carry):
        def slot_vb(b):
            return (ci * _SLOTS + b) * _NW + wid

        def guard(b, fn):
            i_local = ci * _SLOTS + b

            @pl.when(i_local < n_i)
            def _():
                fn()

        for b in range(_SLOTS):
            def fire_in(b=b):
                vb = slot_vb(b)
                for g in range(4):
                    in_copy(b, vb, g).start()
            guard(b, fire_in)
        for b in range(_SLOTS):
            def work(b=b):
                vb = slot_vb(b)
                for g in range(4):
                    in_copy(b, vb, g).wait()
                transpose_slot(b)
                out_copy(b, vb).start()
            guard(b, work)
        for b in range(_SLOTS):
            def drain_out(b=b):
                out_copy(b, slot_vb(b)).wait()
            guard(b, drain_out)
        return carry

    lax.fori_loop(0, n_chunks, chunk_body, 0)


@functools.partial(
    pl.kernel,
    out_type=jax.ShapeDtypeStruct((_VPAD * EMBED,), jnp.float32),
    mesh=plsc.VectorSubcoreMesh(core_axis_name="c", subcore_axis_name="s"),
    compiler_params=pltpu.CompilerParams(
        needs_layout_passes=False, use_tc_tiling_on_sc=True),
    scratch_types=[
        pltpu.VMEM((_SLOTS * 32, _STRIDE), jnp.float32),
        pltpu.VMEM((_SLOTS * 4096,), jnp.float32),
        pltpu.SemaphoreType.DMA,
        pltpu.SemaphoreType.DMA,
    ],
)
def _detile_sc(tblT_hbm, out_hbm, in_v, tr_v, sem_i, sem_o):
    _detile_kernel(tblT_hbm, out_hbm, in_v, tr_v, sem_i, sem_o)


def _gather_kernel(tbl_hbm, idx_t_hbm, idx_c_hbm, w_hbm, b_hbm, out_hbm,
                   idx_t_v, idx_c_v, t_v, c_v, out_v, w_v, b_v,
                   sem_t, sem_c):
    wid = lax.axis_index("s") * _NC + lax.axis_index("c")
    base = wid * _BPW
    pltpu.sync_copy(idx_t_hbm.at[pl.ds(base, _BPW)], idx_t_v)
    pltpu.sync_copy(idx_c_hbm.at[pl.ds(base, _BPW)], idx_c_v)
    pltpu.sync_copy(w_hbm, w_v)
    pltpu.sync_copy(b_hbm, b_v)
    cp_t = pltpu.async_copy(tbl_hbm.at[idx_t_v], t_v, sem_t)
    cp_c = pltpu.async_copy(tbl_hbm.at[idx_c_v], c_v, sem_c)
    cp_t.wait()
    cp_c.wait()

    wv = w_v[...]
    bv = b_v[...]
    lanes = lax.iota(jnp.int32, 16)

    def body(g, carry):
        rows = jnp.int32(g) * 16 + lanes
        acc = jnp.zeros((16,), jnp.float32)
        for e in range(EMBED):
            col = jnp.full((16,), e, jnp.int32)
            tv = plsc.load_gather(t_v, [rows, col])
            cv = plsc.load_gather(c_v, [rows, col])
            acc = acc + tv * cv
        z = acc * wv + bv
        out_v[pl.ds(g * 16, 16)] = 1.0 / (1.0 + jnp.exp(-z))
        return carry

    lax.fori_loop(0, _BPW // 16, body, 0)
    pltpu.sync_copy(out_v, out_hbm.at[pl.ds(base, _BPW)])


@functools.partial(
    pl.kernel,
    out_type=jax.ShapeDtypeStruct((BATCH,), jnp.float32),
    mesh=plsc.VectorSubcoreMesh(core_axis_name="c", subcore_axis_name="s"),
    compiler_params=pltpu.CompilerParams(
        needs_layout_passes=False, use_tc_tiling_on_sc=False),
    scratch_types=[
        pltpu.VMEM((_BPW,), jnp.int32),
        pltpu.VMEM((_BPW,), jnp.int32),
        pltpu.VMEM((_BPW, EMBED), jnp.float32),
        pltpu.VMEM((_BPW, EMBED), jnp.float32),
        pltpu.VMEM((_BPW,), jnp.float32),
        pltpu.VMEM((16,), jnp.float32),
        pltpu.VMEM((16,), jnp.float32),
        pltpu.SemaphoreType.DMA,
        pltpu.SemaphoreType.DMA,
    ],
)
def _context_model_sc(tbl_hbm, idx_t_hbm, idx_c_hbm, w_hbm, b_hbm, out_hbm,
                      idx_t_v, idx_c_v, t_v, c_v, out_v, w_v, b_v,
                      sem_t, sem_c):
    _gather_kernel(tbl_hbm, idx_t_hbm, idx_c_hbm, w_hbm, b_hbm, out_hbm,
                   idx_t_v, idx_c_v, t_v, c_v, out_v, w_v, b_v,
                   sem_t, sem_c)


def kernel(inputs, table, W, b):
    idx_t = inputs[:, 0].astype(jnp.int32)
    idx_c = inputs[:, 1].astype(jnp.int32)
    tbl_vmaj = _detile_sc(table.T).reshape(_VPAD, EMBED)
    w16 = jnp.full((16,), W[0, 0], dtype=jnp.float32)
    b16 = jnp.full((16,), b[0], dtype=jnp.float32)
    out = _context_model_sc(tbl_vmaj, idx_t, idx_c, w16, b16)
    return out.reshape(BATCH, 1)
